# Initial kernel scaffold; baseline (speedup 1.0000x reference)
import jax
import jax.numpy as jnp
from jax.experimental import pallas as pl


def kernel(x, z, pos, edge_index, batch, num_nodes, l, W_emb, b_emb, We1, be1, We2, be2, Wc1, bc1, Wc2):
    raise NotImplementedError("write your pallas kernel here")


# speedup vs baseline: 4.2026x; 4.2026x over previous
import functools

import jax
import jax.numpy as jnp
from jax import lax
from jax.experimental import pallas as pl
from jax.experimental.pallas import tpu as pltpu
from jax.experimental.pallas import tpu_sc as plsc

NC = 2
NS = 16
NW = NC * NS
CH = 80
TW = 48
VW = 8

F32 = jnp.float32
I32 = jnp.int32


def _silu(v):
    return v * (1.0 / (1.0 + jnp.exp(-v)))


def _node_body(x_ref, b_ref, pos_ref, z_ref, wemb_ref, bemb_ref, we1_ref,
               be1_ref, ta_ref, tb_ref):
    z = z_ref[:]
    wemb = wemb_ref[:]
    zw = jnp.dot(z, wemb[1:, :], preferred_element_type=F32)
    bvals = b_ref[:]
    nb = zw.shape[0]
    onehot = (bvals == lax.broadcasted_iota(I32, (bvals.shape[0], nb), 1))
    h = (x_ref[:] * wemb[0:1, :]
         + jnp.dot(onehot.astype(F32), zw, preferred_element_type=F32)
         + bemb_ref[:])
    we1 = we1_ref[:]
    ta_h = jnp.dot(h, we1[:32, :], preferred_element_type=F32) + be1_ref[:]
    tb_h = jnp.dot(h, we1[32:64, :], preferred_element_type=F32)
    p = pos_ref[:][:, :3]
    zpad = jnp.zeros((h.shape[0], TW - 35), dtype=F32)
    ta_ref[:] = jnp.concatenate([ta_h, p, zpad], axis=1)
    tb_ref[:] = jnp.concatenate([tb_h, -p, zpad], axis=1)


def _gather_body(row_hbm, col_hbm, ta_hbm, tb_hbm, out_hbm,
                 idx_r, idx_c, ga, sem):
    n_chunks = row_hbm.shape[0] // (NW * CH)
    wid = lax.axis_index("s") * NC + lax.axis_index("c")
    base = wid * (n_chunks * CH)

    def step(i, carry):
        off = base + i * CH
        pltpu.sync_copy(row_hbm.at[pl.ds(off, CH)], idx_r)
        pltpu.sync_copy(col_hbm.at[pl.ds(off, CH)], idx_c)
        pltpu.async_copy(ta_hbm.at[idx_r], ga, sem).wait()
        pltpu.async_copy(tb_hbm.at[idx_c], ga, sem, add=True).wait()
        pltpu.sync_copy(ga, out_hbm.at[pl.ds(off, CH)])
        return carry

    lax.fori_loop(0, n_chunks, step, 0)


def _mlp_body(g_ref, we1r_ref, we2_ref, be2_ref, wc1_ref, bc1_ref, wc2_ref,
              l_ref, v_ref):
    lv = l_ref[0, 0]
    g = g_ref[:]
    pre = g[:, :32]
    draw = g[:, 32:35]
    d = jnp.where(draw > 0.5 * lv, draw - lv, draw)
    d = jnp.where(d < -0.5 * lv, d + lv, d)
    radial = jnp.sum(d * d, axis=-1, keepdims=True)
    dn = d / (jnp.sqrt(radial) + 1.0)
    m = _silu(pre + radial * we1r_ref[:])
    m = _silu(jnp.dot(m, we2_ref[:], preferred_element_type=F32) + be2_ref[:])
    c = _silu(jnp.dot(m, wc1_ref[:], preferred_element_type=F32) + bc1_ref[:])
    e = jnp.tanh(jnp.dot(c, wc2_ref[:], preferred_element_type=F32)) * 15.0
    out = dn * e
    zpad = jnp.zeros((out.shape[0], VW - 3), dtype=F32)
    v_ref[:] = jnp.concatenate([out, zpad], axis=1)


def _scatter_body(row_hbm, v_hbm, zeros_hbm, p_hbm, idx_r, vbuf, shared):
    npad = zeros_hbm.shape[0]
    rows_per_tile = npad // NS
    cid = lax.axis_index("c")
    sid = lax.axis_index("s")
    wid = sid * NC + cid
    n_chunks = row_hbm.shape[0] // (NW * CH)
    base = wid * (n_chunks * CH)

    pltpu.sync_copy(zeros_hbm.at[pl.ds(sid * rows_per_tile, rows_per_tile)],
                    shared.at[pl.ds(sid * rows_per_tile, rows_per_tile)])
    plsc.subcore_barrier()

    def step(i, carry):
        off = base + i * CH
        pltpu.sync_copy(row_hbm.at[pl.ds(off, CH)], idx_r)
        pltpu.sync_copy(v_hbm.at[pl.ds(off, CH)], vbuf)
        pltpu.sync_copy(vbuf, shared.at[idx_r], add=True)
        return carry

    lax.fori_loop(0, n_chunks, step, 0)
    plsc.subcore_barrier()
    pltpu.sync_copy(shared.at[pl.ds(sid * rows_per_tile, rows_per_tile)],
                    p_hbm.at[cid, pl.ds(sid * rows_per_tile, rows_per_tile)])


def _combine_body(p_ref, l_ref, nn_ref, out_ref):
    lv = l_ref[0, 0]
    i = pl.program_id(0)
    blk = out_ref.shape[0]
    p = p_ref[:]
    s = p[0] + p[1]
    rowid = i * blk + lax.broadcasted_iota(I32, (blk, 1), 0)
    s = jnp.where(rowid < nn_ref[0, 0], s, jnp.zeros_like(s))
    s = jnp.where(s > 0.5 * lv, s - lv, s)
    s = jnp.where(s < -0.5 * lv, s + lv, s)
    out_ref[:] = s


def kernel(x, z, pos, edge_index, batch, num_nodes, l,
           W_emb, b_emb, We1, be1, We2, be2, Wc1, bc1, Wc2):
    n = x.shape[0]
    e = edge_index.shape[1]
    blkn = 1024
    npad = ((n + NS * blkn - 1) // (NS * blkn)) * (NS * blkn)
    echunk = NW * CH
    ep = ((e + echunk - 1) // echunk) * echunk

    x2 = jnp.pad(x.astype(F32), (0, npad - n)).reshape(npad, 1)
    b2 = jnp.pad(batch.astype(I32), (0, npad - n)).reshape(npad, 1)
    pos2 = jnp.pad(pos.astype(F32), ((0, npad - n), (0, 5)))
    row = jnp.pad(edge_index[0].astype(I32), (0, ep - e),
                  constant_values=npad - 1)
    col = jnp.pad(edge_index[1].astype(I32), (0, ep - e))
    l11 = jnp.asarray(l, F32).reshape(1, 1)
    nn11 = jnp.asarray(num_nodes, I32).reshape(1, 1)
    b_emb2 = b_emb.reshape(1, -1).astype(F32)
    be12 = be1.reshape(1, -1).astype(F32)
    be22 = be2.reshape(1, -1).astype(F32)
    bc12 = bc1.reshape(1, -1).astype(F32)
    we1r = We1[64:65, :].astype(F32)

    grid_n = npad // blkn
    ta, tb = pl.pallas_call(
        _node_body,
        grid=(grid_n,),
        in_specs=[
            pl.BlockSpec((blkn, 1), lambda i: (i, 0)),
            pl.BlockSpec((blkn, 1), lambda i: (i, 0)),
            pl.BlockSpec((blkn, 8), lambda i: (i, 0)),
            pl.BlockSpec(z.shape, lambda i: (0, 0)),
            pl.BlockSpec(W_emb.shape, lambda i: (0, 0)),
            pl.BlockSpec((1, 32), lambda i: (0, 0)),
            pl.BlockSpec(We1.shape, lambda i: (0, 0)),
            pl.BlockSpec((1, 32), lambda i: (0, 0)),
        ],
        out_specs=[
            pl.BlockSpec((blkn, TW), lambda i: (i, 0)),
            pl.BlockSpec((blkn, TW), lambda i: (i, 0)),
        ],
        out_shape=[
            jax.ShapeDtypeStruct((npad, TW), F32),
            jax.ShapeDtypeStruct((npad, TW), F32),
        ],
    )(x2, b2, pos2, z.astype(F32), W_emb.astype(F32), b_emb2,
      We1.astype(F32), be12)

    mesh = plsc.VectorSubcoreMesh(core_axis_name="c", subcore_axis_name="s")
    gather = pl.kernel(
        _gather_body,
        out_type=jax.ShapeDtypeStruct((ep, TW), F32),
        mesh=mesh,
        compiler_params=pltpu.CompilerParams(use_tc_tiling_on_sc=False),
        scratch_types=[
            pltpu.VMEM((CH,), I32),
            pltpu.VMEM((CH,), I32),
            pltpu.VMEM((CH, TW), F32),
            pltpu.SemaphoreType.DMA,
        ],
    )
    g = gather(row, col, ta, tb)

    blke = 4000
    egrid = ep // blke if ep % blke == 0 else ep // (NW * CH)
    if ep % blke != 0:
        blke = NW * CH
        egrid = ep // blke
    v = pl.pallas_call(
        _mlp_body,
        grid=(egrid,),
        in_specs=[
            pl.BlockSpec((blke, TW), lambda i: (i, 0)),
            pl.BlockSpec((1, 32), lambda i: (0, 0)),
            pl.BlockSpec(We2.shape, lambda i: (0, 0)),
            pl.BlockSpec((1, 32), lambda i: (0, 0)),
            pl.BlockSpec(Wc1.shape, lambda i: (0, 0)),
            pl.BlockSpec((1, 32), lambda i: (0, 0)),
            pl.BlockSpec(Wc2.shape, lambda i: (0, 0)),
            pl.BlockSpec(memory_space=pltpu.SMEM),
        ],
        out_specs=pl.BlockSpec((blke, VW), lambda i: (i, 0)),
        out_shape=jax.ShapeDtypeStruct((ep, VW), F32),
    )(g, we1r, We2.astype(F32), be22, Wc1.astype(F32), bc12,
      Wc2.astype(F32), l11)

    zeros_tab = jnp.zeros((npad, VW), F32)
    scatter = pl.kernel(
        _scatter_body,
        out_type=jax.ShapeDtypeStruct((NC, npad, VW), F32),
        mesh=plsc.VectorSubcoreMesh(core_axis_name="c", subcore_axis_name="s"),
        compiler_params=pltpu.CompilerParams(use_tc_tiling_on_sc=False),
        scratch_types=[
            pltpu.VMEM((CH,), I32),
            pltpu.VMEM((CH, VW), F32),
            pltpu.VMEM_SHARED((npad, VW), F32),
        ],
    )
    p = scatter(row, v, zeros_tab)

    diff = pl.pallas_call(
        _combine_body,
        grid=(npad // blkn,),
        in_specs=[
            pl.BlockSpec((NC, blkn, VW), lambda i: (0, i, 0)),
            pl.BlockSpec(memory_space=pltpu.SMEM),
            pl.BlockSpec(memory_space=pltpu.SMEM),
        ],
        out_specs=pl.BlockSpec((blkn, VW), lambda i: (i, 0)),
        out_shape=jax.ShapeDtypeStruct((npad, VW), F32),
    )(p, l11, nn11)

    return diff[:n, :3]


# speedup vs baseline: 6.2223x; 1.4806x over previous
import functools

import jax
import jax.numpy as jnp
from jax import lax
from jax.experimental import pallas as pl
from jax.experimental.pallas import tpu as pltpu
from jax.experimental.pallas import tpu_sc as plsc

NC = 2
NS = 16
NW = NC * NS
CH = 1000
TW = 48
VW = 8

F32 = jnp.float32
I32 = jnp.int32


def _silu(v):
    return v * (1.0 / (1.0 + jnp.exp(-v)))


def _node_body(x_ref, b_ref, pos_ref, z_ref, wemb_ref, bemb_ref, we1_ref,
               be1_ref, ta_ref, tb_ref):
    z = z_ref[:]
    wemb = wemb_ref[:]
    zw = jnp.dot(z, wemb[1:, :], preferred_element_type=F32)
    bvals = b_ref[:]
    nb = zw.shape[0]
    onehot = (bvals == lax.broadcasted_iota(I32, (bvals.shape[0], nb), 1))
    h = (x_ref[:] * wemb[0:1, :]
         + jnp.dot(onehot.astype(F32), zw, preferred_element_type=F32)
         + bemb_ref[:])
    we1 = we1_ref[:]
    ta_h = jnp.dot(h, we1[:32, :], preferred_element_type=F32) + be1_ref[:]
    tb_h = jnp.dot(h, we1[32:64, :], preferred_element_type=F32)
    p = pos_ref[:][:, :3]
    zpad = jnp.zeros((h.shape[0], TW - 35), dtype=F32)
    ta_ref[:] = jnp.concatenate([ta_h, p, zpad], axis=1)
    tb_ref[:] = jnp.concatenate([tb_h, -p, zpad], axis=1)


def _gather_body(ei_hbm, ta_hbm, tb_hbm, out_hbm,
                 idx_r, idx_c, ga, sem):
    n_chunks = ei_hbm.shape[1] // (NW * CH)
    wid = lax.axis_index("s") * NC + lax.axis_index("c")
    base = wid * (n_chunks * CH)

    def step(i, carry):
        off = base + i * CH
        pltpu.sync_copy(ei_hbm.at[0, pl.ds(off, CH)], idx_r)
        pltpu.sync_copy(ei_hbm.at[1, pl.ds(off, CH)], idx_c)
        pltpu.async_copy(ta_hbm.at[idx_r], ga, sem).wait()
        pltpu.async_copy(tb_hbm.at[idx_c], ga, sem, add=True).wait()
        pltpu.sync_copy(ga, out_hbm.at[pl.ds(off, CH)])
        return carry

    lax.fori_loop(0, n_chunks, step, 0)


def _mlp_body(g_ref, we1r_ref, we2_ref, be2_ref, wc1_ref, bc1_ref, wc2_ref,
              l_ref, v_ref):
    lv = l_ref[0, 0]
    g = g_ref[:]
    pre = g[:, :32]
    draw = g[:, 32:35]
    d = jnp.where(draw > 0.5 * lv, draw - lv, draw)
    d = jnp.where(d < -0.5 * lv, d + lv, d)
    radial = jnp.sum(d * d, axis=-1, keepdims=True)
    dn = d / (jnp.sqrt(radial) + 1.0)
    m = _silu(pre + radial * we1r_ref[:])
    m = _silu(jnp.dot(m, we2_ref[:], preferred_element_type=F32) + be2_ref[:])
    c = _silu(jnp.dot(m, wc1_ref[:], preferred_element_type=F32) + bc1_ref[:])
    e = jnp.tanh(jnp.dot(c, wc2_ref[:], preferred_element_type=F32)) * 15.0
    out = dn * e
    zpad = jnp.zeros((out.shape[0], VW - 3), dtype=F32)
    v_ref[:] = jnp.concatenate([out, zpad], axis=1)


def _scatter_body(ei_hbm, v_hbm, zeros_hbm, p_hbm, idx_r, vbuf, shared):
    npad = zeros_hbm.shape[0]
    rows_per_tile = npad // NS
    cid = lax.axis_index("c")
    sid = lax.axis_index("s")
    wid = sid * NC + cid
    n_chunks = v_hbm.shape[0] // (NW * CH)
    base = wid * (n_chunks * CH)

    pltpu.sync_copy(zeros_hbm.at[pl.ds(sid * rows_per_tile, rows_per_tile)],
                    shared.at[pl.ds(sid * rows_per_tile, rows_per_tile)])
    plsc.subcore_barrier()

    def step(i, carry):
        off = base + i * CH
        pltpu.sync_copy(ei_hbm.at[0, pl.ds(off, CH)], idx_r)
        pltpu.sync_copy(v_hbm.at[pl.ds(off, CH)], vbuf)
        pltpu.sync_copy(vbuf, shared.at[idx_r], add=True)
        return carry

    lax.fori_loop(0, n_chunks, step, 0)
    plsc.subcore_barrier()
    pltpu.sync_copy(shared.at[pl.ds(sid * rows_per_tile, rows_per_tile)],
                    p_hbm.at[cid, pl.ds(sid * rows_per_tile, rows_per_tile)])


def _combine_body(p_ref, l_ref, nn_ref, out_ref):
    lv = l_ref[0, 0]
    i = pl.program_id(0)
    blk = out_ref.shape[0]
    p = p_ref[:]
    s = p[0] + p[1]
    rowid = i * blk + lax.broadcasted_iota(I32, (blk, 1), 0)
    s = jnp.where(rowid < nn_ref[0, 0], s, jnp.zeros_like(s))
    s = jnp.where(s > 0.5 * lv, s - lv, s)
    s = jnp.where(s < -0.5 * lv, s + lv, s)
    out_ref[:] = s


def kernel(x, z, pos, edge_index, batch, num_nodes, l,
           W_emb, b_emb, We1, be1, We2, be2, Wc1, bc1, Wc2):
    n = x.shape[0]
    e = edge_index.shape[1]
    blkn = 1024
    npad = ((n + NS * blkn - 1) // (NS * blkn)) * (NS * blkn)
    echunk = NW * CH
    ep = ((e + echunk - 1) // echunk) * echunk

    x2 = jnp.pad(x.astype(F32), (0, npad - n)).reshape(npad, 1)
    b2 = jnp.pad(batch.astype(I32), (0, npad - n)).reshape(npad, 1)
    pos2 = jnp.pad(pos.astype(F32), ((0, npad - n), (0, 5)))
    ei = edge_index.astype(I32)
    if ep != e:
        ei = jnp.concatenate(
            [ei, jnp.full((2, ep - e), npad - 1, I32)], axis=1)
    l11 = jnp.asarray(l, F32).reshape(1, 1)
    nn11 = jnp.asarray(num_nodes, I32).reshape(1, 1)
    b_emb2 = b_emb.reshape(1, -1).astype(F32)
    be12 = be1.reshape(1, -1).astype(F32)
    be22 = be2.reshape(1, -1).astype(F32)
    bc12 = bc1.reshape(1, -1).astype(F32)
    we1r = We1[64:65, :].astype(F32)

    grid_n = npad // blkn
    ta, tb = pl.pallas_call(
        _node_body,
        grid=(grid_n,),
        in_specs=[
            pl.BlockSpec((blkn, 1), lambda i: (i, 0)),
            pl.BlockSpec((blkn, 1), lambda i: (i, 0)),
            pl.BlockSpec((blkn, 8), lambda i: (i, 0)),
            pl.BlockSpec(z.shape, lambda i: (0, 0)),
            pl.BlockSpec(W_emb.shape, lambda i: (0, 0)),
            pl.BlockSpec((1, 32), lambda i: (0, 0)),
            pl.BlockSpec(We1.shape, lambda i: (0, 0)),
            pl.BlockSpec((1, 32), lambda i: (0, 0)),
        ],
        out_specs=[
            pl.BlockSpec((blkn, TW), lambda i: (i, 0)),
            pl.BlockSpec((blkn, TW), lambda i: (i, 0)),
        ],
        out_shape=[
            jax.ShapeDtypeStruct((npad, TW), F32),
            jax.ShapeDtypeStruct((npad, TW), F32),
        ],
    )(x2, b2, pos2, z.astype(F32), W_emb.astype(F32), b_emb2,
      We1.astype(F32), be12)

    mesh = plsc.VectorSubcoreMesh(core_axis_name="c", subcore_axis_name="s")
    gather = pl.kernel(
        _gather_body,
        out_type=jax.ShapeDtypeStruct((ep, TW), F32),
        mesh=mesh,
        compiler_params=pltpu.CompilerParams(use_tc_tiling_on_sc=False),
        scratch_types=[
            pltpu.VMEM((CH,), I32),
            pltpu.VMEM((CH,), I32),
            pltpu.VMEM((CH, TW), F32),
            pltpu.SemaphoreType.DMA,
        ],
    )
    g = gather(ei, ta, tb)

    blke = 4000
    egrid = ep // blke if ep % blke == 0 else ep // (NW * CH)
    if ep % blke != 0:
        blke = NW * CH
        egrid = ep // blke
    v = pl.pallas_call(
        _mlp_body,
        grid=(egrid,),
        in_specs=[
            pl.BlockSpec((blke, TW), lambda i: (i, 0)),
            pl.BlockSpec((1, 32), lambda i: (0, 0)),
            pl.BlockSpec(We2.shape, lambda i: (0, 0)),
            pl.BlockSpec((1, 32), lambda i: (0, 0)),
            pl.BlockSpec(Wc1.shape, lambda i: (0, 0)),
            pl.BlockSpec((1, 32), lambda i: (0, 0)),
            pl.BlockSpec(Wc2.shape, lambda i: (0, 0)),
            pl.BlockSpec(memory_space=pltpu.SMEM),
        ],
        out_specs=pl.BlockSpec((blke, VW), lambda i: (i, 0)),
        out_shape=jax.ShapeDtypeStruct((ep, VW), F32),
    )(g, we1r, We2.astype(F32), be22, Wc1.astype(F32), bc12,
      Wc2.astype(F32), l11)

    zeros_tab = jnp.zeros((npad, VW), F32)
    scatter = pl.kernel(
        _scatter_body,
        out_type=jax.ShapeDtypeStruct((NC, npad, VW), F32),
        mesh=plsc.VectorSubcoreMesh(core_axis_name="c", subcore_axis_name="s"),
        compiler_params=pltpu.CompilerParams(use_tc_tiling_on_sc=False),
        scratch_types=[
            pltpu.VMEM((CH,), I32),
            pltpu.VMEM((CH, VW), F32),
            pltpu.VMEM_SHARED((npad, VW), F32),
        ],
    )
    p = scatter(ei, v, zeros_tab)

    diff = pl.pallas_call(
        _combine_body,
        grid=(npad // blkn,),
        in_specs=[
            pl.BlockSpec((NC, blkn, VW), lambda i: (0, i, 0)),
            pl.BlockSpec(memory_space=pltpu.SMEM),
            pl.BlockSpec(memory_space=pltpu.SMEM),
        ],
        out_specs=pl.BlockSpec((blkn, VW), lambda i: (i, 0)),
        out_shape=jax.ShapeDtypeStruct((npad, VW), F32),
    )(p, l11, nn11)

    return diff[:n, :3]


# speedup vs baseline: 6.8858x; 1.1066x over previous
import functools

import jax
import jax.numpy as jnp
from jax import lax
from jax.experimental import pallas as pl
from jax.experimental.pallas import tpu as pltpu
from jax.experimental.pallas import tpu_sc as plsc

NC = 2
NS = 16
NW = NC * NS
CH = 1000
TW = 48
VW = 8

F32 = jnp.float32
I32 = jnp.int32


def _silu(v):
    return 0.5 * v * (1.0 + jnp.tanh(0.5 * v))


def _node_body(x_ref, b_ref, pos_ref, z_ref, wemb_ref, bemb_ref, we1_ref,
               be1_ref, ta_ref, tb_ref):
    z = z_ref[:]
    wemb = wemb_ref[:]
    zw = jnp.dot(z, wemb[1:, :], preferred_element_type=F32)
    bvals = b_ref[:]
    nb = zw.shape[0]
    onehot = (bvals == lax.broadcasted_iota(I32, (bvals.shape[0], nb), 1))
    h = (x_ref[:] * wemb[0:1, :]
         + jnp.dot(onehot.astype(F32), zw, preferred_element_type=F32)
         + bemb_ref[:])
    we1 = we1_ref[:]
    ta_h = jnp.dot(h, we1[:32, :], preferred_element_type=F32) + be1_ref[:]
    tb_h = jnp.dot(h, we1[32:64, :], preferred_element_type=F32)
    p = pos_ref[:][:, :3]
    zpad = jnp.zeros((h.shape[0], TW - 35), dtype=F32)
    ta_ref[:] = jnp.concatenate([ta_h, p, zpad], axis=1)
    tb_ref[:] = jnp.concatenate([tb_h, -p, zpad], axis=1)


def _gather_body(ei_hbm, ta_hbm, tb_hbm, out_hbm,
                 idx_r, idx_c, ga, sem):
    n_chunks = ei_hbm.shape[1] // (NW * CH)
    wid = lax.axis_index("s") * NC + lax.axis_index("c")
    base = wid * (n_chunks * CH)

    def step(i, carry):
        off = base + i * CH
        pltpu.sync_copy(ei_hbm.at[0, pl.ds(off, CH)], idx_r)
        pltpu.sync_copy(ei_hbm.at[1, pl.ds(off, CH)], idx_c)
        pltpu.async_copy(ta_hbm.at[idx_r], ga, sem).wait()
        pltpu.async_copy(tb_hbm.at[idx_c], ga, sem, add=True).wait()
        pltpu.sync_copy(ga, out_hbm.at[pl.ds(off, CH), pl.ds(0, TW)])
        return carry

    lax.fori_loop(0, n_chunks, step, 0)


def _mlp_body(g_ref, we1r_ref, ssel_ref, w2b_ref, be2b_ref, w1b_ref,
              bc1b_ref, wq_ref, l_ref, vl_ref, vr_ref):
    lv = l_ref[0, 0]
    g = g_ref[:]
    lane = lax.broadcasted_iota(I32, g.shape, 1)
    gz = jnp.where(jnp.bitwise_and(lane, 63) < 48, g, jnp.zeros_like(g))
    dw = jnp.where(gz > 0.5 * lv, gz - lv, gz)
    dw = jnp.where(dw < -0.5 * lv, dw + lv, dw)
    rb = jnp.dot(dw * dw, ssel_ref[:], preferred_element_type=F32,
                 precision=lax.Precision.HIGHEST)
    t = gz + rb * we1r_ref[:]
    m = _silu(t)
    m = _silu(jnp.dot(m, w2b_ref[:], preferred_element_type=F32)
              + be2b_ref[:])
    c = _silu(jnp.dot(m, w1b_ref[:], preferred_element_type=F32)
              + bc1b_ref[:])
    qb = jnp.dot(c, wq_ref[:], preferred_element_type=F32,
                 precision=lax.Precision.HIGHEST)
    s = jnp.tanh(qb) * 15.0 / (jnp.sqrt(rb) + 1.0)
    out = dw * s
    vl_ref[:] = out[:, 32:32 + VW]
    vr_ref[:] = out[:, 96:96 + VW]


def _scatter_body(rowe_hbm, rowo_hbm, vl_hbm, vr_hbm, zeros_hbm, p_hbm,
                  idx_r, vbuf, shared):
    npad = zeros_hbm.shape[0]
    rows_per_tile = npad // NS
    cid = lax.axis_index("c")
    sid = lax.axis_index("s")
    wid = sid * NC + cid
    n_chunks = vl_hbm.shape[0] // (NW * CH)
    base = wid * (n_chunks * CH)

    pltpu.sync_copy(zeros_hbm.at[pl.ds(sid * rows_per_tile, rows_per_tile)],
                    shared.at[pl.ds(sid * rows_per_tile, rows_per_tile)])
    plsc.subcore_barrier()

    def make_step(row_hbm, v_hbm):
        def step(i, carry):
            off = base + i * CH
            pltpu.sync_copy(row_hbm.at[pl.ds(off, CH)], idx_r)
            pltpu.sync_copy(v_hbm.at[pl.ds(off, CH)], vbuf)
            pltpu.sync_copy(vbuf, shared.at[idx_r], add=True)
            return carry
        return step

    lax.fori_loop(0, n_chunks, make_step(rowe_hbm, vl_hbm), 0)
    lax.fori_loop(0, n_chunks, make_step(rowo_hbm, vr_hbm), 0)
    plsc.subcore_barrier()
    pltpu.sync_copy(shared.at[pl.ds(sid * rows_per_tile, rows_per_tile)],
                    p_hbm.at[cid, pl.ds(sid * rows_per_tile, rows_per_tile)])


def _combine_body(p_ref, l_ref, nn_ref, out_ref):
    lv = l_ref[0, 0]
    i = pl.program_id(0)
    blk = out_ref.shape[0]
    p = p_ref[:]
    s = p[0] + p[1]
    rowid = i * blk + lax.broadcasted_iota(I32, (blk, 1), 0)
    s = jnp.where(rowid < nn_ref[0, 0], s, jnp.zeros_like(s))
    s = jnp.where(s > 0.5 * lv, s - lv, s)
    s = jnp.where(s < -0.5 * lv, s + lv, s)
    out_ref[:] = s


def kernel(x, z, pos, edge_index, batch, num_nodes, l,
           W_emb, b_emb, We1, be1, We2, be2, Wc1, bc1, Wc2):
    n = x.shape[0]
    e = edge_index.shape[1]
    blkn = 1024
    npad = ((n + NS * blkn - 1) // (NS * blkn)) * (NS * blkn)
    echunk = 2 * NW * CH
    ep = ((e + echunk - 1) // echunk) * echunk

    x2 = jnp.pad(x.astype(F32), (0, npad - n)).reshape(npad, 1)
    b2 = jnp.pad(batch.astype(I32), (0, npad - n)).reshape(npad, 1)
    pos2 = jnp.pad(pos.astype(F32), ((0, npad - n), (0, 5)))
    ei = edge_index.astype(I32)
    if ep != e:
        ei = jnp.concatenate(
            [ei, jnp.full((2, ep - e), npad - 1, I32)], axis=1)
    l11 = jnp.asarray(l, F32).reshape(1, 1)
    nn11 = jnp.asarray(num_nodes, I32).reshape(1, 1)
    b_emb2 = b_emb.reshape(1, -1).astype(F32)
    be12 = be1.reshape(1, -1).astype(F32)
    be22 = be2.reshape(1, -1).astype(F32)
    bc12 = bc1.reshape(1, -1).astype(F32)
    we1r = We1[64:65, :].astype(F32)

    grid_n = npad // blkn
    ta, tb = pl.pallas_call(
        _node_body,
        grid=(grid_n,),
        in_specs=[
            pl.BlockSpec((blkn, 1), lambda i: (i, 0)),
            pl.BlockSpec((blkn, 1), lambda i: (i, 0)),
            pl.BlockSpec((blkn, 8), lambda i: (i, 0)),
            pl.BlockSpec(z.shape, lambda i: (0, 0)),
            pl.BlockSpec(W_emb.shape, lambda i: (0, 0)),
            pl.BlockSpec((1, 32), lambda i: (0, 0)),
            pl.BlockSpec(We1.shape, lambda i: (0, 0)),
            pl.BlockSpec((1, 32), lambda i: (0, 0)),
        ],
        out_specs=[
            pl.BlockSpec((blkn, TW), lambda i: (i, 0)),
            pl.BlockSpec((blkn, TW), lambda i: (i, 0)),
        ],
        out_shape=[
            jax.ShapeDtypeStruct((npad, TW), F32),
            jax.ShapeDtypeStruct((npad, TW), F32),
        ],
    )(x2, b2, pos2, z.astype(F32), W_emb.astype(F32), b_emb2,
      We1.astype(F32), be12)

    mesh = plsc.VectorSubcoreMesh(core_axis_name="c", subcore_axis_name="s")
    gather = pl.kernel(
        _gather_body,
        out_type=jax.ShapeDtypeStruct((ep, 64), F32),
        mesh=mesh,
        compiler_params=pltpu.CompilerParams(use_tc_tiling_on_sc=False),
        scratch_types=[
            pltpu.VMEM((CH,), I32),
            pltpu.VMEM((CH,), I32),
            pltpu.VMEM((CH, TW), F32),
            pltpu.SemaphoreType.DMA,
        ],
    )
    g = gather(ei, ta, tb)

    ep2 = ep // 2
    g2 = g.reshape(ep2, 128)
    blk2 = 2000
    egrid = ep2 // blk2
    zw = jnp.zeros((32, 32), F32)
    zb = jnp.zeros((1, 32), F32)
    z16 = jnp.zeros((1, 16), F32)
    we1r2 = jnp.concatenate(
        [we1r, jnp.zeros((1, 32), F32), we1r, jnp.zeros((1, 32), F32)], 1)
    lane = jnp.arange(128)
    half = lane < 64
    dlane = (jnp.bitwise_and(lane, 63) >= 32) & (jnp.bitwise_and(lane, 63) < 35)
    ssel = (dlane[:, None] & (half[:, None] == half[None, :])).astype(F32)
    w2b = jnp.block([[We2.astype(F32), zw, zw, zw],
                     [zw, zw, zw, zw],
                     [zw, zw, We2.astype(F32), zw],
                     [zw, zw, zw, zw]])
    be2b = jnp.concatenate([be22, zb, be22, zb], 1)
    w1b = jnp.block([[Wc1.astype(F32), zw, zw, zw],
                     [zw, zw, zw, zw],
                     [zw, zw, Wc1.astype(F32), zw],
                     [zw, zw, zw, zw]])
    bc1b = jnp.concatenate([bc12, zb, bc12, zb], 1)
    hidlane = (jnp.bitwise_and(lane, 63) < 32)
    wq_col = jnp.where(hidlane, jnp.pad(jnp.tile(Wc2.astype(F32)[:, 0], 4),
                                        (0, 0)), 0.0)
    wq = (wq_col[:, None] * (half[:, None] == half[None, :]).astype(F32))
    vl, vr = pl.pallas_call(
        _mlp_body,
        grid=(egrid,),
        in_specs=[
            pl.BlockSpec((blk2, 128), lambda i: (i, 0)),
            pl.BlockSpec((1, 128), lambda i: (0, 0)),
            pl.BlockSpec((128, 128), lambda i: (0, 0)),
            pl.BlockSpec((128, 128), lambda i: (0, 0)),
            pl.BlockSpec((1, 128), lambda i: (0, 0)),
            pl.BlockSpec((128, 128), lambda i: (0, 0)),
            pl.BlockSpec((1, 128), lambda i: (0, 0)),
            pl.BlockSpec((128, 128), lambda i: (0, 0)),
            pl.BlockSpec(memory_space=pltpu.SMEM),
        ],
        out_specs=[
            pl.BlockSpec((blk2, VW), lambda i: (i, 0)),
            pl.BlockSpec((blk2, VW), lambda i: (i, 0)),
        ],
        out_shape=[
            jax.ShapeDtypeStruct((ep2, VW), F32),
            jax.ShapeDtypeStruct((ep2, VW), F32),
        ],
    )(g2, we1r2, ssel, w2b, be2b, w1b, bc1b, wq, l11)

    rowe = ei[0, 0::2]
    rowo = ei[0, 1::2]
    zeros_tab = jnp.zeros((npad, VW), F32)
    scatter = pl.kernel(
        _scatter_body,
        out_type=jax.ShapeDtypeStruct((NC, npad, VW), F32),
        mesh=plsc.VectorSubcoreMesh(core_axis_name="c", subcore_axis_name="s"),
        compiler_params=pltpu.CompilerParams(use_tc_tiling_on_sc=False),
        scratch_types=[
            pltpu.VMEM((CH,), I32),
            pltpu.VMEM((CH, VW), F32),
            pltpu.VMEM_SHARED((npad, VW), F32),
        ],
    )
    p = scatter(rowe, rowo, vl, vr, zeros_tab)

    diff = pl.pallas_call(
        _combine_body,
        grid=(npad // blkn,),
        in_specs=[
            pl.BlockSpec((NC, blkn, VW), lambda i: (0, i, 0)),
            pl.BlockSpec(memory_space=pltpu.SMEM),
            pl.BlockSpec(memory_space=pltpu.SMEM),
        ],
        out_specs=pl.BlockSpec((blkn, VW), lambda i: (i, 0)),
        out_shape=jax.ShapeDtypeStruct((npad, VW), F32),
    )(p, l11, nn11)

    return diff[:n, :3]


# speedup vs baseline: 8.3396x; 1.2111x over previous
import jax
import jax.numpy as jnp
from jax import lax
from jax.experimental import pallas as pl
from jax.experimental.pallas import tpu as pltpu
from jax.experimental.pallas import tpu_sc as plsc

NC = 2
NS = 16
NW = NC * NS
CHG = 2000
CHS = 1000
TW = 48
VW = 8

F32 = jnp.float32
I32 = jnp.int32


def _silu(v):
    return 0.5 * v * (1.0 + jnp.tanh(0.5 * v))


def _node_body(x_ref, b_ref, pos_ref, z_ref, wemb_ref, bemb_ref, we1_ref,
               be1_ref, ta_ref, tb_ref):
    z = z_ref[:]
    wemb = wemb_ref[:]
    zw = jnp.dot(z, wemb[1:, :], preferred_element_type=F32)
    bvals = b_ref[:]
    nb = zw.shape[0]
    onehot = (bvals == lax.broadcasted_iota(I32, (bvals.shape[0], nb), 1))
    h = (x_ref[:] * wemb[0:1, :]
         + jnp.dot(onehot.astype(F32), zw, preferred_element_type=F32)
         + bemb_ref[:])
    we1 = we1_ref[:]
    ta_h = jnp.dot(h, we1[:32, :], preferred_element_type=F32) + be1_ref[:]
    tb_h = jnp.dot(h, we1[32:64, :], preferred_element_type=F32)
    p = pos_ref[:][:, :3]
    zpad = jnp.zeros((h.shape[0], TW - 35), dtype=F32)
    ta_ref[:] = jnp.concatenate([ta_h, p, zpad], axis=1)
    tb_ref[:] = jnp.concatenate([tb_h, -p, zpad], axis=1)


def _gather_body(ei_hbm, ta_hbm, tb_hbm, lvec_hbm, out_hbm,
                 idx_r, idx_c, ga, lv_v, sem):
    e_tot = ei_hbm.shape[1]
    ep2 = e_tot // 2
    ec = e_tot // NW
    n_chunks = ec // CHG
    wid = lax.axis_index("s") * NC + lax.axis_index("c")
    half = wid // (NW // 2)
    colw = half * TW

    pltpu.sync_copy(lvec_hbm, lv_v)
    lv = lv_v[...]
    iota = lax.iota(I32, 16)

    def col(cc):
        return jnp.full((16,), cc, I32)

    def step(i, carry):
        eoff = wid * ec + i * CHG
        row = eoff - half * ep2
        pltpu.sync_copy(ei_hbm.at[0, pl.ds(eoff, CHG)], idx_r)
        pltpu.sync_copy(ei_hbm.at[1, pl.ds(eoff, CHG)], idx_c)
        pltpu.async_copy(ta_hbm.at[idx_r], ga, sem).wait()
        pltpu.async_copy(tb_hbm.at[idx_c], ga, sem, add=True).wait()

        def wrap_radial(j, c2):
            rows = j * 16 + iota
            d0 = plsc.load_gather(ga, [rows, col(32)])
            d1 = plsc.load_gather(ga, [rows, col(33)])
            d2 = plsc.load_gather(ga, [rows, col(34)])
            d0 = jnp.where(d0 > 0.5 * lv, d0 - lv, d0)
            d0 = jnp.where(d0 < -0.5 * lv, d0 + lv, d0)
            d1 = jnp.where(d1 > 0.5 * lv, d1 - lv, d1)
            d1 = jnp.where(d1 < -0.5 * lv, d1 + lv, d1)
            d2 = jnp.where(d2 > 0.5 * lv, d2 - lv, d2)
            d2 = jnp.where(d2 < -0.5 * lv, d2 + lv, d2)
            r = d0 * d0 + d1 * d1 + d2 * d2
            plsc.store_scatter(ga, [rows, col(32)], d0)
            plsc.store_scatter(ga, [rows, col(33)], d1)
            plsc.store_scatter(ga, [rows, col(34)], d2)
            plsc.store_scatter(ga, [rows, col(35)], r)
            plsc.store_scatter(ga, [rows, col(36)], r)
            plsc.store_scatter(ga, [rows, col(37)], r)
            plsc.store_scatter(ga, [rows, col(38)], r)
            return c2

        lax.fori_loop(0, CHG // 16, wrap_radial, 0)
        pltpu.sync_copy(ga, out_hbm.at[pl.ds(row, CHG), pl.ds(colw, TW)])
        return carry

    lax.fori_loop(0, n_chunks, step, 0)


def _mlp_body(g_ref, sr_ref, w2b_ref, be2b_ref, w1b_ref, bc1b_ref, wq2_ref,
              vl_ref, vr_ref):
    g = g_ref[:]
    t = g + jnp.dot(g, sr_ref[:], preferred_element_type=F32)
    m = _silu(t)
    m = _silu(jnp.dot(m, w2b_ref[:], preferred_element_type=F32)
              + be2b_ref[:])
    c = _silu(jnp.dot(m, w1b_ref[:], preferred_element_type=F32)
              + bc1b_ref[:])
    qb = jnp.dot(c, wq2_ref[:], preferred_element_type=F32)
    rba = pltpu.roll(g, 93, 1)
    s = jnp.tanh(qb) * 15.0 / (jnp.sqrt(rba) + 1.0)
    out = g * s
    vl_ref[:] = out[:, 32:32 + VW]
    vr_ref[:] = out[:, 80:80 + VW]


def _scatter_body(ei_hbm, vl_hbm, vr_hbm, zeros_hbm, p_hbm,
                  idx_r, vbuf, shared):
    npad = zeros_hbm.shape[0]
    rows_per_tile = npad // NS
    cid = lax.axis_index("c")
    sid = lax.axis_index("s")
    wid = sid * NC + cid
    ep2 = vl_hbm.shape[0]
    ecs = ep2 // NW
    n_chunks = ecs // CHS
    base = wid * ecs

    pltpu.sync_copy(zeros_hbm.at[pl.ds(sid * rows_per_tile, rows_per_tile)],
                    shared.at[pl.ds(sid * rows_per_tile, rows_per_tile)])
    plsc.subcore_barrier()

    def make_step(v_hbm, ibase):
        def step(i, carry):
            off = base + i * CHS
            pltpu.sync_copy(ei_hbm.at[0, pl.ds(ibase + off, CHS)], idx_r)
            pltpu.sync_copy(v_hbm.at[pl.ds(off, CHS)], vbuf)
            pltpu.sync_copy(vbuf, shared.at[idx_r], add=True)
            return carry
        return step

    lax.fori_loop(0, n_chunks, make_step(vl_hbm, 0), 0)
    lax.fori_loop(0, n_chunks, make_step(vr_hbm, ep2), 0)
    plsc.subcore_barrier()
    pltpu.sync_copy(shared.at[pl.ds(sid * rows_per_tile, rows_per_tile)],
                    p_hbm.at[cid, pl.ds(sid * rows_per_tile, rows_per_tile)])


def _combine_body(p_ref, l_ref, nn_ref, out_ref):
    lv = l_ref[0, 0]
    i = pl.program_id(0)
    blk = out_ref.shape[0]
    p = p_ref[:]
    s = p[0] + p[1]
    rowid = i * blk + lax.broadcasted_iota(I32, (blk, 1), 0)
    s = jnp.where(rowid < nn_ref[0, 0], s, jnp.zeros_like(s))
    s = jnp.where(s > 0.5 * lv, s - lv, s)
    s = jnp.where(s < -0.5 * lv, s + lv, s)
    out_ref[:] = s


def kernel(x, z, pos, edge_index, batch, num_nodes, l,
           W_emb, b_emb, We1, be1, We2, be2, Wc1, bc1, Wc2):
    n = x.shape[0]
    e = edge_index.shape[1]
    blkn = 1024
    npad = ((n + NS * blkn - 1) // (NS * blkn)) * (NS * blkn)
    echunk = NW * CHG
    ep = ((e + echunk - 1) // echunk) * echunk
    ep2 = ep // 2

    x2 = jnp.pad(x.astype(F32), (0, npad - n)).reshape(npad, 1)
    b2 = jnp.pad(batch.astype(I32), (0, npad - n)).reshape(npad, 1)
    pos2 = jnp.pad(pos.astype(F32), ((0, npad - n), (0, 5)))
    ei = edge_index.astype(I32)
    if ep != e:
        ei = jnp.concatenate(
            [ei, jnp.full((2, ep - e), npad - 1, I32)], axis=1)
    l11 = jnp.asarray(l, F32).reshape(1, 1)
    lvec = jnp.full((16,), l, F32)
    nn11 = jnp.asarray(num_nodes, I32).reshape(1, 1)
    b_emb2 = b_emb.reshape(1, -1).astype(F32)
    be12 = be1.reshape(1, -1).astype(F32)
    we1r = We1[64, :].astype(F32)

    grid_n = npad // blkn
    ta, tb = pl.pallas_call(
        _node_body,
        grid=(grid_n,),
        in_specs=[
            pl.BlockSpec((blkn, 1), lambda i: (i, 0)),
            pl.BlockSpec((blkn, 1), lambda i: (i, 0)),
            pl.BlockSpec((blkn, 8), lambda i: (i, 0)),
            pl.BlockSpec(z.shape, lambda i: (0, 0)),
            pl.BlockSpec(W_emb.shape, lambda i: (0, 0)),
            pl.BlockSpec((1, 32), lambda i: (0, 0)),
            pl.BlockSpec(We1.shape, lambda i: (0, 0)),
            pl.BlockSpec((1, 32), lambda i: (0, 0)),
        ],
        out_specs=[
            pl.BlockSpec((blkn, TW), lambda i: (i, 0)),
            pl.BlockSpec((blkn, TW), lambda i: (i, 0)),
        ],
        out_shape=[
            jax.ShapeDtypeStruct((npad, TW), F32),
            jax.ShapeDtypeStruct((npad, TW), F32),
        ],
    )(x2, b2, pos2, z.astype(F32), W_emb.astype(F32), b_emb2,
      We1.astype(F32), be12)

    mesh = plsc.VectorSubcoreMesh(core_axis_name="c", subcore_axis_name="s")
    gather = pl.kernel(
        _gather_body,
        out_type=jax.ShapeDtypeStruct((ep2, 2 * TW), F32),
        mesh=mesh,
        compiler_params=pltpu.CompilerParams(use_tc_tiling_on_sc=False,
                                             needs_layout_passes=False),
        scratch_types=[
            pltpu.VMEM((CHG,), I32),
            pltpu.VMEM((CHG,), I32),
            pltpu.VMEM((CHG, TW), F32),
            pltpu.VMEM((16,), F32),
            pltpu.SemaphoreType.DMA,
        ],
    )
    g2 = gather(ei, ta, tb, lvec)

    blk2 = 2000
    egrid = ep2 // blk2
    We2f = We2.astype(F32)
    Wc1f = Wc1.astype(F32)
    Wc2f = Wc2.astype(F32)
    sr = (jnp.zeros((96, 96), F32)
          .at[35, 0:32].set(we1r).at[83, 48:80].set(we1r))
    w2b = (jnp.zeros((96, 96), F32)
           .at[0:32, 0:32].set(We2f).at[48:80, 48:80].set(We2f))
    be2b = (jnp.zeros((1, 96), F32)
            .at[0, 0:32].set(be2.astype(F32))
            .at[0, 48:80].set(be2.astype(F32)))
    w1b = (jnp.zeros((96, 96), F32)
           .at[0:32, 0:32].set(Wc1f).at[48:80, 48:80].set(Wc1f))
    bc1b = (jnp.zeros((1, 96), F32)
            .at[0, 0:32].set(bc1.astype(F32))
            .at[0, 48:80].set(bc1.astype(F32)))
    wq2 = (jnp.zeros((96, 96), F32)
           .at[0:32, 32:35].set(jnp.tile(Wc2f, (1, 3)))
           .at[48:80, 80:83].set(jnp.tile(Wc2f, (1, 3))))
    vl, vr = pl.pallas_call(
        _mlp_body,
        grid=(egrid,),
        in_specs=[
            pl.BlockSpec((blk2, 96), lambda i: (i, 0)),
            pl.BlockSpec((96, 96), lambda i: (0, 0)),
            pl.BlockSpec((96, 96), lambda i: (0, 0)),
            pl.BlockSpec((1, 96), lambda i: (0, 0)),
            pl.BlockSpec((96, 96), lambda i: (0, 0)),
            pl.BlockSpec((1, 96), lambda i: (0, 0)),
            pl.BlockSpec((96, 96), lambda i: (0, 0)),
        ],
        out_specs=[
            pl.BlockSpec((blk2, VW), lambda i: (i, 0)),
            pl.BlockSpec((blk2, VW), lambda i: (i, 0)),
        ],
        out_shape=[
            jax.ShapeDtypeStruct((ep2, VW), F32),
            jax.ShapeDtypeStruct((ep2, VW), F32),
        ],
    )(g2, sr, w2b, be2b, w1b, bc1b, wq2)

    zeros_tab = jnp.zeros((npad, VW), F32)
    scatter = pl.kernel(
        _scatter_body,
        out_type=jax.ShapeDtypeStruct((NC, npad, VW), F32),
        mesh=plsc.VectorSubcoreMesh(core_axis_name="c", subcore_axis_name="s"),
        compiler_params=pltpu.CompilerParams(use_tc_tiling_on_sc=False),
        scratch_types=[
            pltpu.VMEM((CHS,), I32),
            pltpu.VMEM((CHS, VW), F32),
            pltpu.VMEM_SHARED((npad, VW), F32),
        ],
    )
    p = scatter(ei, vl, vr, zeros_tab)

    diff = pl.pallas_call(
        _combine_body,
        grid=(npad // blkn,),
        in_specs=[
            pl.BlockSpec((NC, blkn, VW), lambda i: (0, i, 0)),
            pl.BlockSpec(memory_space=pltpu.SMEM),
            pl.BlockSpec(memory_space=pltpu.SMEM),
        ],
        out_specs=pl.BlockSpec((blkn, VW), lambda i: (i, 0)),
        out_shape=jax.ShapeDtypeStruct((npad, VW), F32),
    )(p, l11, nn11)

    return diff[:n, :3]


# speedup vs baseline: 8.5851x; 1.0294x over previous
import jax
import jax.numpy as jnp
from jax import lax
from jax.experimental import pallas as pl
from jax.experimental.pallas import tpu as pltpu
from jax.experimental.pallas import tpu_sc as plsc

NC = 2
NS = 16
NW = NC * NS
CHG = 2000
CHS = 1000
TW = 48
VW = 8

F32 = jnp.float32
I32 = jnp.int32


def _silu(v):
    return 0.5 * v * (1.0 + jnp.tanh(0.5 * v))


def _node_body(x_ref, b_ref, pos_ref, z_ref, wemb_ref, bemb_ref, we1_ref,
               be1_ref, ta_ref, tb_ref):
    z = z_ref[:]
    wemb = wemb_ref[:]
    zw = jnp.dot(z, wemb[1:, :], preferred_element_type=F32)
    bvals = b_ref[:]
    nb = zw.shape[0]
    onehot = (bvals == lax.broadcasted_iota(I32, (bvals.shape[0], nb), 1))
    h = (x_ref[:] * wemb[0:1, :]
         + jnp.dot(onehot.astype(F32), zw, preferred_element_type=F32)
         + bemb_ref[:])
    we1 = we1_ref[:]
    ta_h = jnp.dot(h, we1[:32, :], preferred_element_type=F32) + be1_ref[:]
    tb_h = jnp.dot(h, we1[32:64, :], preferred_element_type=F32)
    p = pos_ref[:]
    zpad = jnp.zeros((h.shape[0], TW - 35), dtype=F32)
    ta_ref[:] = jnp.concatenate([ta_h, p, zpad], axis=1)
    tb_ref[:] = jnp.concatenate([tb_h, -p, zpad], axis=1)


def _gather_body(ei_hbm, ta_hbm, tb_hbm, lvec_hbm, out_hbm,
                 idx_r, idx_c, ga, lv_v, sem):
    e_tot = ei_hbm.shape[1]
    ep2 = e_tot // 2
    ec = e_tot // NW
    n_chunks = ec // CHG
    wid = lax.axis_index("s") * NC + lax.axis_index("c")
    half = wid // (NW // 2)
    colw = half * TW

    pltpu.sync_copy(lvec_hbm, lv_v)
    lv = lv_v[...]
    iota = lax.iota(I32, 16)

    def col(cc):
        return jnp.full((16,), cc, I32)

    def step(i, carry):
        eoff = wid * ec + i * CHG
        row = eoff - half * ep2
        pltpu.sync_copy(ei_hbm.at[0, pl.ds(eoff, CHG)], idx_r)
        pltpu.sync_copy(ei_hbm.at[1, pl.ds(eoff, CHG)], idx_c)
        pltpu.async_copy(ta_hbm.at[idx_r], ga, sem).wait()
        pltpu.async_copy(tb_hbm.at[idx_c], ga, sem, add=True).wait()

        def wrap_radial(j, c2):
            rows = j * 16 + iota
            d0 = plsc.load_gather(ga, [rows, col(32)])
            d1 = plsc.load_gather(ga, [rows, col(33)])
            d2 = plsc.load_gather(ga, [rows, col(34)])
            d0 = jnp.where(d0 > 0.5 * lv, d0 - lv, d0)
            d0 = jnp.where(d0 < -0.5 * lv, d0 + lv, d0)
            d1 = jnp.where(d1 > 0.5 * lv, d1 - lv, d1)
            d1 = jnp.where(d1 < -0.5 * lv, d1 + lv, d1)
            d2 = jnp.where(d2 > 0.5 * lv, d2 - lv, d2)
            d2 = jnp.where(d2 < -0.5 * lv, d2 + lv, d2)
            r = d0 * d0 + d1 * d1 + d2 * d2
            plsc.store_scatter(ga, [rows, col(32)], d0)
            plsc.store_scatter(ga, [rows, col(33)], d1)
            plsc.store_scatter(ga, [rows, col(34)], d2)
            plsc.store_scatter(ga, [rows, col(35)], r)
            plsc.store_scatter(ga, [rows, col(36)], r)
            plsc.store_scatter(ga, [rows, col(37)], r)
            plsc.store_scatter(ga, [rows, col(38)], r)
            return c2

        lax.fori_loop(0, CHG // 16, wrap_radial, 0)
        pltpu.sync_copy(ga, out_hbm.at[pl.ds(row, CHG), pl.ds(colw, TW)])
        return carry

    lax.fori_loop(0, n_chunks, step, 0)


def _mlp_body(g_ref, sr_ref, w2b_ref, be2b_ref, w1b_ref, bc1b_ref, wq2_ref,
              vl_ref, vr_ref):
    g = g_ref[:]
    t = g + jnp.dot(g, sr_ref[:], preferred_element_type=F32)
    m = _silu(t)
    m = _silu(jnp.dot(m, w2b_ref[:], preferred_element_type=F32)
              + be2b_ref[:])
    c = _silu(jnp.dot(m, w1b_ref[:], preferred_element_type=F32)
              + bc1b_ref[:])
    qb = jnp.dot(c, wq2_ref[:], preferred_element_type=F32)
    rba = pltpu.roll(g, 93, 1)
    s = jnp.tanh(qb) * 15.0 / (jnp.sqrt(rba) + 1.0)
    out = g * s
    vl_ref[:] = out[:, 32:32 + VW]
    vr_ref[:] = out[:, 80:80 + VW]


def _scatter_body(ei_hbm, vl_hbm, vr_hbm, zeros_hbm, p_hbm,
                  idx_r, vbuf, shared):
    npad = zeros_hbm.shape[0]
    rows_per_tile = npad // NS
    cid = lax.axis_index("c")
    sid = lax.axis_index("s")
    wid = sid * NC + cid
    ep2 = vl_hbm.shape[0]
    ecs = ep2 // NW
    n_chunks = ecs // CHS
    base = wid * ecs

    pltpu.sync_copy(zeros_hbm.at[pl.ds(sid * rows_per_tile, rows_per_tile)],
                    shared.at[pl.ds(sid * rows_per_tile, rows_per_tile)])
    plsc.subcore_barrier()

    def make_step(v_hbm, ibase):
        def step(i, carry):
            off = base + i * CHS
            pltpu.sync_copy(ei_hbm.at[0, pl.ds(ibase + off, CHS)], idx_r)
            pltpu.sync_copy(v_hbm.at[pl.ds(off, CHS)], vbuf)
            pltpu.sync_copy(vbuf, shared.at[idx_r], add=True)
            return carry
        return step

    lax.fori_loop(0, n_chunks, make_step(vl_hbm, 0), 0)
    lax.fori_loop(0, n_chunks, make_step(vr_hbm, ep2), 0)
    plsc.subcore_barrier()
    pltpu.sync_copy(shared.at[pl.ds(sid * rows_per_tile, rows_per_tile)],
                    p_hbm.at[cid, pl.ds(sid * rows_per_tile, rows_per_tile)])


def _combine_body(p_ref, l_ref, nn_ref, out_ref):
    lv = l_ref[0, 0]
    i = pl.program_id(0)
    blk = out_ref.shape[0]
    p = p_ref[:]
    s = p[0] + p[1]
    rowid = i * blk + lax.broadcasted_iota(I32, (blk, 1), 0)
    s = jnp.where(rowid < nn_ref[0, 0], s, jnp.zeros_like(s))
    s = jnp.where(s > 0.5 * lv, s - lv, s)
    s = jnp.where(s < -0.5 * lv, s + lv, s)
    out_ref[:] = s[:, :3]


def kernel(x, z, pos, edge_index, batch, num_nodes, l,
           W_emb, b_emb, We1, be1, We2, be2, Wc1, bc1, Wc2):
    n = x.shape[0]
    e = edge_index.shape[1]
    blkn = 1000
    lnode = 2000
    echunk = NW * CHG
    ep = ((e + echunk - 1) // echunk) * echunk
    ep2 = ep // 2
    if ep == e and n % lnode == 0:
        npad = n
    else:
        npad = (n // lnode + 1) * lnode

    if npad == n:
        x2 = x.astype(F32).reshape(n, 1)
        b2 = batch.astype(I32).reshape(n, 1)
        pos2 = pos.astype(F32)
    else:
        x2 = jnp.pad(x.astype(F32), (0, npad - n)).reshape(npad, 1)
        b2 = jnp.pad(batch.astype(I32), (0, npad - n)).reshape(npad, 1)
        pos2 = jnp.pad(pos.astype(F32), ((0, npad - n), (0, 0)))
    ei = edge_index.astype(I32)
    if ep != e:
        ei = jnp.concatenate(
            [ei, jnp.full((2, ep - e), npad - 1, I32)], axis=1)
    l11 = jnp.asarray(l, F32).reshape(1, 1)
    lvec = jnp.full((16,), l, F32)
    nn11 = jnp.asarray(num_nodes, I32).reshape(1, 1)
    b_emb2 = b_emb.reshape(1, -1).astype(F32)
    be12 = be1.reshape(1, -1).astype(F32)
    we1r = We1[64, :].astype(F32)

    grid_n = npad // blkn
    ta, tb = pl.pallas_call(
        _node_body,
        grid=(grid_n,),
        in_specs=[
            pl.BlockSpec((blkn, 1), lambda i: (i, 0)),
            pl.BlockSpec((blkn, 1), lambda i: (i, 0)),
            pl.BlockSpec((blkn, 3), lambda i: (i, 0)),
            pl.BlockSpec(z.shape, lambda i: (0, 0)),
            pl.BlockSpec(W_emb.shape, lambda i: (0, 0)),
            pl.BlockSpec((1, 32), lambda i: (0, 0)),
            pl.BlockSpec(We1.shape, lambda i: (0, 0)),
            pl.BlockSpec((1, 32), lambda i: (0, 0)),
        ],
        out_specs=[
            pl.BlockSpec((blkn, TW), lambda i: (i, 0)),
            pl.BlockSpec((blkn, TW), lambda i: (i, 0)),
        ],
        out_shape=[
            jax.ShapeDtypeStruct((npad, TW), F32),
            jax.ShapeDtypeStruct((npad, TW), F32),
        ],
    )(x2, b2, pos2, z.astype(F32), W_emb.astype(F32), b_emb2,
      We1.astype(F32), be12)

    mesh = plsc.VectorSubcoreMesh(core_axis_name="c", subcore_axis_name="s")
    gather = pl.kernel(
        _gather_body,
        out_type=jax.ShapeDtypeStruct((ep2, 2 * TW), F32),
        mesh=mesh,
        compiler_params=pltpu.CompilerParams(use_tc_tiling_on_sc=False,
                                             needs_layout_passes=False),
        scratch_types=[
            pltpu.VMEM((CHG,), I32),
            pltpu.VMEM((CHG,), I32),
            pltpu.VMEM((CHG, TW), F32),
            pltpu.VMEM((16,), F32),
            pltpu.SemaphoreType.DMA,
        ],
    )
    g2 = gather(ei, ta, tb, lvec)

    blk2 = 2000
    egrid = ep2 // blk2
    We2f = We2.astype(F32)
    Wc1f = Wc1.astype(F32)
    Wc2f = Wc2.astype(F32)
    sr = (jnp.zeros((96, 96), F32)
          .at[35, 0:32].set(we1r).at[83, 48:80].set(we1r))
    w2b = (jnp.zeros((96, 96), F32)
           .at[0:32, 0:32].set(We2f).at[48:80, 48:80].set(We2f))
    be2b = (jnp.zeros((1, 96), F32)
            .at[0, 0:32].set(be2.astype(F32))
            .at[0, 48:80].set(be2.astype(F32)))
    w1b = (jnp.zeros((96, 96), F32)
           .at[0:32, 0:32].set(Wc1f).at[48:80, 48:80].set(Wc1f))
    bc1b = (jnp.zeros((1, 96), F32)
            .at[0, 0:32].set(bc1.astype(F32))
            .at[0, 48:80].set(bc1.astype(F32)))
    wq2 = (jnp.zeros((96, 96), F32)
           .at[0:32, 32:35].set(jnp.tile(Wc2f, (1, 3)))
           .at[48:80, 80:83].set(jnp.tile(Wc2f, (1, 3))))
    vl, vr = pl.pallas_call(
        _mlp_body,
        grid=(egrid,),
        in_specs=[
            pl.BlockSpec((blk2, 96), lambda i: (i, 0)),
            pl.BlockSpec((96, 96), lambda i: (0, 0)),
            pl.BlockSpec((96, 96), lambda i: (0, 0)),
            pl.BlockSpec((1, 96), lambda i: (0, 0)),
            pl.BlockSpec((96, 96), lambda i: (0, 0)),
            pl.BlockSpec((1, 96), lambda i: (0, 0)),
            pl.BlockSpec((96, 96), lambda i: (0, 0)),
        ],
        out_specs=[
            pl.BlockSpec((blk2, VW), lambda i: (i, 0)),
            pl.BlockSpec((blk2, VW), lambda i: (i, 0)),
        ],
        out_shape=[
            jax.ShapeDtypeStruct((ep2, VW), F32),
            jax.ShapeDtypeStruct((ep2, VW), F32),
        ],
    )(g2, sr, w2b, be2b, w1b, bc1b, wq2)

    zeros_tab = jnp.zeros((npad, VW), F32)
    scatter = pl.kernel(
        _scatter_body,
        out_type=jax.ShapeDtypeStruct((NC, npad, VW), F32),
        mesh=plsc.VectorSubcoreMesh(core_axis_name="c", subcore_axis_name="s"),
        compiler_params=pltpu.CompilerParams(use_tc_tiling_on_sc=False),
        scratch_types=[
            pltpu.VMEM((CHS,), I32),
            pltpu.VMEM((CHS, VW), F32),
            pltpu.VMEM_SHARED((npad, VW), F32),
        ],
    )
    p = scatter(ei, vl, vr, zeros_tab)

    diff = pl.pallas_call(
        _combine_body,
        grid=(npad // blkn,),
        in_specs=[
            pl.BlockSpec((NC, blkn, VW), lambda i: (0, i, 0)),
            pl.BlockSpec(memory_space=pltpu.SMEM),
            pl.BlockSpec(memory_space=pltpu.SMEM),
        ],
        out_specs=pl.BlockSpec((blkn, 3), lambda i: (i, 0)),
        out_shape=jax.ShapeDtypeStruct((npad, 3), F32),
    )(p, l11, nn11)

    return diff if npad == n else diff[:n]


# speedup vs baseline: 10.1313x; 1.1801x over previous
import jax
import jax.numpy as jnp
from jax import lax
from jax.experimental import pallas as pl
from jax.experimental.pallas import tpu as pltpu
from jax.experimental.pallas import tpu_sc as plsc

NC = 2
NS = 16
NW = NC * NS
CHG = 2000
CHS = 1000
TW = 48
VW = 8

F32 = jnp.float32
I32 = jnp.int32


def _silu(v):
    return 0.5 * v * (1.0 + jnp.tanh(0.5 * v))


def _node_body(x_ref, b_ref, pos_ref, z_ref, wemb_ref, bemb_ref, we1_ref,
               be1_ref, ta_ref, tb_ref):
    z = z_ref[:]
    wemb = wemb_ref[:]
    zw = jnp.dot(z, wemb[1:, :], preferred_element_type=F32)
    bvals = b_ref[:]
    nb = zw.shape[0]
    onehot = (bvals == lax.broadcasted_iota(I32, (bvals.shape[0], nb), 1))
    h = (x_ref[:] * wemb[0:1, :]
         + jnp.dot(onehot.astype(F32), zw, preferred_element_type=F32)
         + bemb_ref[:])
    we1 = we1_ref[:]
    ta_h = jnp.dot(h, we1[:32, :], preferred_element_type=F32) + be1_ref[:]
    tb_h = jnp.dot(h, we1[32:64, :], preferred_element_type=F32)
    p = pos_ref[:]
    zpad = jnp.zeros((h.shape[0], TW - 35), dtype=F32)
    ta_ref[:] = jnp.concatenate([ta_h, p, zpad], axis=1)
    tb_ref[:] = jnp.concatenate([tb_h, -p, zpad], axis=1)


def _gather_body(ei_hbm, ta_hbm, tb_hbm, lvec_hbm, out_hbm,
                 idx_r, idx_c, ga, lv_v, sem):
    e_tot = ei_hbm.shape[1]
    ep2 = e_tot // 2
    ec = e_tot // NW
    n_chunks = ec // CHG
    wid = lax.axis_index("s") * NC + lax.axis_index("c")
    half = wid // (NW // 2)
    colw = half * 64

    pltpu.sync_copy(lvec_hbm, lv_v)
    lv = lv_v[...]
    iota = lax.iota(I32, 16)

    def col(cc):
        return jnp.full((16,), cc, I32)

    def step(i, carry):
        eoff = wid * ec + i * CHG
        row = eoff - half * ep2
        pltpu.sync_copy(ei_hbm.at[0, pl.ds(eoff, CHG)], idx_r)
        pltpu.sync_copy(ei_hbm.at[1, pl.ds(eoff, CHG)], idx_c)
        pltpu.async_copy(ta_hbm.at[idx_r], ga, sem).wait()
        pltpu.async_copy(tb_hbm.at[idx_c], ga, sem, add=True).wait()

        def wrap_radial(j, c2):
            rows = j * 16 + iota
            d0 = plsc.load_gather(ga, [rows, col(32)])
            d1 = plsc.load_gather(ga, [rows, col(33)])
            d2 = plsc.load_gather(ga, [rows, col(34)])
            d0 = jnp.where(d0 > 0.5 * lv, d0 - lv, d0)
            d0 = jnp.where(d0 < -0.5 * lv, d0 + lv, d0)
            d1 = jnp.where(d1 > 0.5 * lv, d1 - lv, d1)
            d1 = jnp.where(d1 < -0.5 * lv, d1 + lv, d1)
            d2 = jnp.where(d2 > 0.5 * lv, d2 - lv, d2)
            d2 = jnp.where(d2 < -0.5 * lv, d2 + lv, d2)
            r = d0 * d0 + d1 * d1 + d2 * d2
            plsc.store_scatter(ga, [rows, col(32)], d0)
            plsc.store_scatter(ga, [rows, col(33)], d1)
            plsc.store_scatter(ga, [rows, col(34)], d2)
            plsc.store_scatter(ga, [rows, col(35)], r)
            plsc.store_scatter(ga, [rows, col(36)], r)
            plsc.store_scatter(ga, [rows, col(37)], r)
            plsc.store_scatter(ga, [rows, col(38)], r)
            return c2

        lax.fori_loop(0, CHG // 16, wrap_radial, 0)
        pltpu.sync_copy(ga, out_hbm.at[pl.ds(row, CHG), pl.ds(colw, TW)])
        return carry

    lax.fori_loop(0, n_chunks, step, 0)


def _mlp_body(g_ref, sr_ref, w2b_ref, be2b_ref, w1b_ref, bc1b_ref, wq2_ref,
              vl_ref, vr_ref):
    graw = g_ref[:]
    lane = lax.broadcasted_iota(I32, graw.shape, 1)
    g = jnp.where(jnp.bitwise_and(lane, 63) < TW, graw, jnp.zeros_like(graw))
    t = g + jnp.dot(g, sr_ref[:], preferred_element_type=F32)
    m = _silu(t)
    m = _silu(jnp.dot(m, w2b_ref[:], preferred_element_type=F32)
              + be2b_ref[:])
    c = _silu(jnp.dot(m, w1b_ref[:], preferred_element_type=F32)
              + bc1b_ref[:])
    qb = jnp.dot(c, wq2_ref[:], preferred_element_type=F32)
    rba = pltpu.roll(g, 125, 1)
    s = jnp.tanh(qb) * 15.0 / (jnp.sqrt(rba) + 1.0)
    out = g * s
    vl_ref[:] = out[:, 32:32 + VW]
    vr_ref[:] = out[:, 96:96 + VW]


def _scatter_body(ei_hbm, vl_hbm, vr_hbm, zeros_hbm, p_hbm,
                  idx_r, vbuf, shared):
    npad = zeros_hbm.shape[0]
    rows_per_tile = npad // NS
    cid = lax.axis_index("c")
    sid = lax.axis_index("s")
    wid = sid * NC + cid
    ep2 = vl_hbm.shape[0]
    ecs = ep2 // NW
    n_chunks = ecs // CHS
    base = wid * ecs

    pltpu.sync_copy(zeros_hbm.at[pl.ds(sid * rows_per_tile, rows_per_tile)],
                    shared.at[pl.ds(sid * rows_per_tile, rows_per_tile)])
    plsc.subcore_barrier()

    def make_step(v_hbm, ibase):
        def step(i, carry):
            off = base + i * CHS
            pltpu.sync_copy(ei_hbm.at[0, pl.ds(ibase + off, CHS)], idx_r)
            pltpu.sync_copy(v_hbm.at[pl.ds(off, CHS)], vbuf)
            pltpu.sync_copy(vbuf, shared.at[idx_r], add=True)
            return carry
        return step

    lax.fori_loop(0, n_chunks, make_step(vl_hbm, 0), 0)
    lax.fori_loop(0, n_chunks, make_step(vr_hbm, ep2), 0)
    plsc.subcore_barrier()
    pltpu.sync_copy(shared.at[pl.ds(sid * rows_per_tile, rows_per_tile)],
                    p_hbm.at[cid, pl.ds(sid * rows_per_tile, rows_per_tile)])


def _combine_body(p_ref, l_ref, nn_ref, out_ref):
    lv = l_ref[0, 0]
    i = pl.program_id(0)
    blk = out_ref.shape[0]
    p = p_ref[:]
    s = p[0] + p[1]
    rowid = i * blk + lax.broadcasted_iota(I32, (blk, 1), 0)
    s = jnp.where(rowid < nn_ref[0, 0], s, jnp.zeros_like(s))
    s = jnp.where(s > 0.5 * lv, s - lv, s)
    s = jnp.where(s < -0.5 * lv, s + lv, s)
    out_ref[:] = s[:, :3]


def kernel(x, z, pos, edge_index, batch, num_nodes, l,
           W_emb, b_emb, We1, be1, We2, be2, Wc1, bc1, Wc2):
    n = x.shape[0]
    e = edge_index.shape[1]
    blkn = 1000
    lnode = 2000
    echunk = NW * CHG
    ep = ((e + echunk - 1) // echunk) * echunk
    ep2 = ep // 2
    if ep == e and n % lnode == 0:
        npad = n
    else:
        npad = (n // lnode + 1) * lnode

    if npad == n:
        x2 = x.astype(F32).reshape(n, 1)
        b2 = batch.astype(I32).reshape(n, 1)
        pos2 = pos.astype(F32)
    else:
        x2 = jnp.pad(x.astype(F32), (0, npad - n)).reshape(npad, 1)
        b2 = jnp.pad(batch.astype(I32), (0, npad - n)).reshape(npad, 1)
        pos2 = jnp.pad(pos.astype(F32), ((0, npad - n), (0, 0)))
    ei = edge_index.astype(I32)
    if ep != e:
        ei = jnp.concatenate(
            [ei, jnp.full((2, ep - e), npad - 1, I32)], axis=1)
    l11 = jnp.asarray(l, F32).reshape(1, 1)
    lvec = jnp.full((16,), l, F32)
    nn11 = jnp.asarray(num_nodes, I32).reshape(1, 1)
    b_emb2 = b_emb.reshape(1, -1).astype(F32)
    be12 = be1.reshape(1, -1).astype(F32)
    we1r = We1[64, :].astype(F32)

    grid_n = npad // blkn
    ta, tb = pl.pallas_call(
        _node_body,
        grid=(grid_n,),
        in_specs=[
            pl.BlockSpec((blkn, 1), lambda i: (i, 0)),
            pl.BlockSpec((blkn, 1), lambda i: (i, 0)),
            pl.BlockSpec((blkn, 3), lambda i: (i, 0)),
            pl.BlockSpec(z.shape, lambda i: (0, 0)),
            pl.BlockSpec(W_emb.shape, lambda i: (0, 0)),
            pl.BlockSpec((1, 32), lambda i: (0, 0)),
            pl.BlockSpec(We1.shape, lambda i: (0, 0)),
            pl.BlockSpec((1, 32), lambda i: (0, 0)),
        ],
        out_specs=[
            pl.BlockSpec((blkn, TW), lambda i: (i, 0)),
            pl.BlockSpec((blkn, TW), lambda i: (i, 0)),
        ],
        out_shape=[
            jax.ShapeDtypeStruct((npad, TW), F32),
            jax.ShapeDtypeStruct((npad, TW), F32),
        ],
    )(x2, b2, pos2, z.astype(F32), W_emb.astype(F32), b_emb2,
      We1.astype(F32), be12)

    mesh = plsc.VectorSubcoreMesh(core_axis_name="c", subcore_axis_name="s")
    gather = pl.kernel(
        _gather_body,
        out_type=jax.ShapeDtypeStruct((ep2, 128), F32),
        mesh=mesh,
        compiler_params=pltpu.CompilerParams(use_tc_tiling_on_sc=False,
                                             needs_layout_passes=False),
        scratch_types=[
            pltpu.VMEM((CHG,), I32),
            pltpu.VMEM((CHG,), I32),
            pltpu.VMEM((CHG, TW), F32),
            pltpu.VMEM((16,), F32),
            pltpu.SemaphoreType.DMA,
        ],
    )
    g2 = gather(ei, ta, tb, lvec)

    blk2 = 2000
    egrid = ep2 // blk2
    We2f = We2.astype(F32)
    Wc1f = Wc1.astype(F32)
    Wc2f = Wc2.astype(F32)
    sr = (jnp.zeros((128, 128), F32)
          .at[35, 0:32].set(we1r).at[99, 64:96].set(we1r))
    w2b = (jnp.zeros((128, 128), F32)
           .at[0:32, 0:32].set(We2f).at[64:96, 64:96].set(We2f))
    be2b = (jnp.zeros((1, 128), F32)
            .at[0, 0:32].set(be2.astype(F32))
            .at[0, 64:96].set(be2.astype(F32)))
    w1b = (jnp.zeros((128, 128), F32)
           .at[0:32, 0:32].set(Wc1f).at[64:96, 64:96].set(Wc1f))
    bc1b = (jnp.zeros((1, 128), F32)
            .at[0, 0:32].set(bc1.astype(F32))
            .at[0, 64:96].set(bc1.astype(F32)))
    wq2 = (jnp.zeros((128, 128), F32)
           .at[0:32, 32:35].set(jnp.tile(Wc2f, (1, 3)))
           .at[64:96, 96:99].set(jnp.tile(Wc2f, (1, 3))))
    vl, vr = pl.pallas_call(
        _mlp_body,
        grid=(egrid,),
        in_specs=[
            pl.BlockSpec((blk2, 128), lambda i: (i, 0)),
            pl.BlockSpec((128, 128), lambda i: (0, 0)),
            pl.BlockSpec((128, 128), lambda i: (0, 0)),
            pl.BlockSpec((1, 128), lambda i: (0, 0)),
            pl.BlockSpec((128, 128), lambda i: (0, 0)),
            pl.BlockSpec((1, 128), lambda i: (0, 0)),
            pl.BlockSpec((128, 128), lambda i: (0, 0)),
        ],
        out_specs=[
            pl.BlockSpec((blk2, VW), lambda i: (i, 0)),
            pl.BlockSpec((blk2, VW), lambda i: (i, 0)),
        ],
        out_shape=[
            jax.ShapeDtypeStruct((ep2, VW), F32),
            jax.ShapeDtypeStruct((ep2, VW), F32),
        ],
    )(g2, sr, w2b, be2b, w1b, bc1b, wq2)

    zeros_tab = jnp.zeros((npad, VW), F32)
    scatter = pl.kernel(
        _scatter_body,
        out_type=jax.ShapeDtypeStruct((NC, npad, VW), F32),
        mesh=plsc.VectorSubcoreMesh(core_axis_name="c", subcore_axis_name="s"),
        compiler_params=pltpu.CompilerParams(use_tc_tiling_on_sc=False),
        scratch_types=[
            pltpu.VMEM((CHS,), I32),
            pltpu.VMEM((CHS, VW), F32),
            pltpu.VMEM_SHARED((npad, VW), F32),
        ],
    )
    p = scatter(ei, vl, vr, zeros_tab)

    diff = pl.pallas_call(
        _combine_body,
        grid=(npad // blkn,),
        in_specs=[
            pl.BlockSpec((NC, blkn, VW), lambda i: (0, i, 0)),
            pl.BlockSpec(memory_space=pltpu.SMEM),
            pl.BlockSpec(memory_space=pltpu.SMEM),
        ],
        out_specs=pl.BlockSpec((blkn, 3), lambda i: (i, 0)),
        out_shape=jax.ShapeDtypeStruct((npad, 3), F32),
    )(p, l11, nn11)

    return diff if npad == n else diff[:n]


# speedup vs baseline: 10.5388x; 1.0402x over previous
import jax
import jax.numpy as jnp
from jax import lax
from jax.experimental import pallas as pl
from jax.experimental.pallas import tpu as pltpu
from jax.experimental.pallas import tpu_sc as plsc

NC = 2
NS = 16
NW = NC * NS
CHG = 2000
CHS = 1000
TW = 48
VW = 8

F32 = jnp.float32
I32 = jnp.int32


def _silu(v):
    return 0.5 * v * (1.0 + jnp.tanh(0.5 * v))


def _node_body(x_ref, b_ref, pos_ref, z_ref, wemb_ref, bemb_ref, we1_ref,
               be1_ref, ta_ref, tb_ref):
    z = z_ref[:]
    wemb = wemb_ref[:]
    zw = jnp.dot(z, wemb[1:, :], preferred_element_type=F32)
    bvals = b_ref[:]
    nb = zw.shape[0]
    onehot = (bvals == lax.broadcasted_iota(I32, (bvals.shape[0], nb), 1))
    h = (x_ref[:] * wemb[0:1, :]
         + jnp.dot(onehot.astype(F32), zw, preferred_element_type=F32)
         + bemb_ref[:])
    we1 = we1_ref[:]
    ta_h = jnp.dot(h, we1[:32, :], preferred_element_type=F32) + be1_ref[:]
    tb_h = jnp.dot(h, we1[32:64, :], preferred_element_type=F32)
    p = pos_ref[:]
    zpad = jnp.zeros((h.shape[0], TW - 35), dtype=F32)
    ta_ref[:] = jnp.concatenate([ta_h, p, zpad], axis=1)
    tb_ref[:] = jnp.concatenate([tb_h, -p, zpad], axis=1)


def _gather_body(ei_hbm, ta_hbm, tb_hbm, lvec_hbm, out_hbm,
                 idx_r, idx_c, ga, lv_v, sem):
    e_tot = ei_hbm.shape[1]
    ep2 = e_tot // 2
    ec = e_tot // NW
    n_chunks = ec // CHG
    wid = lax.axis_index("s") * NC + lax.axis_index("c")
    half = wid // (NW // 2)
    colw = half * 64

    pltpu.sync_copy(lvec_hbm, lv_v)
    lv = lv_v[...]
    iota = lax.iota(I32, 16)

    def col(cc):
        return jnp.full((16,), cc, I32)

    def step(i, carry):
        eoff = wid * ec + i * CHG
        row = eoff - half * ep2
        pltpu.sync_copy(ei_hbm.at[0, pl.ds(eoff, CHG)], idx_r)
        pltpu.sync_copy(ei_hbm.at[1, pl.ds(eoff, CHG)], idx_c)
        pltpu.async_copy(ta_hbm.at[idx_r], ga, sem).wait()
        pltpu.async_copy(tb_hbm.at[idx_c], ga, sem, add=True).wait()

        def wrap_radial(j, c2):
            rows = j * 16 + iota
            d0 = plsc.load_gather(ga, [rows, col(32)])
            d1 = plsc.load_gather(ga, [rows, col(33)])
            d2 = plsc.load_gather(ga, [rows, col(34)])
            d0 = jnp.where(d0 > 0.5 * lv, d0 - lv, d0)
            d0 = jnp.where(d0 < -0.5 * lv, d0 + lv, d0)
            d1 = jnp.where(d1 > 0.5 * lv, d1 - lv, d1)
            d1 = jnp.where(d1 < -0.5 * lv, d1 + lv, d1)
            d2 = jnp.where(d2 > 0.5 * lv, d2 - lv, d2)
            d2 = jnp.where(d2 < -0.5 * lv, d2 + lv, d2)
            r = d0 * d0 + d1 * d1 + d2 * d2
            plsc.store_scatter(ga, [rows, col(32)], d0)
            plsc.store_scatter(ga, [rows, col(33)], d1)
            plsc.store_scatter(ga, [rows, col(34)], d2)
            plsc.store_scatter(ga, [rows, col(35)], r)
            plsc.store_scatter(ga, [rows, col(36)], r)
            plsc.store_scatter(ga, [rows, col(37)], r)
            plsc.store_scatter(ga, [rows, col(38)], r)
            return c2

        lax.fori_loop(0, CHG // 16, wrap_radial, 0)
        pltpu.sync_copy(ga, out_hbm.at[pl.ds(row, CHG), pl.ds(colw, TW)])
        return carry

    lax.fori_loop(0, n_chunks, step, 0)


def _mlp_body(g_ref, sr_ref, w2b_ref, be2b_ref, w1b_ref, bc1b_ref, wq2_ref,
              vl_ref, vr_ref):
    graw = g_ref[:]
    lane = lax.broadcasted_iota(I32, graw.shape, 1)
    g = jnp.where(jnp.bitwise_and(lane, 63) < TW, graw, jnp.zeros_like(graw))
    t = g + jnp.dot(g, sr_ref[:], preferred_element_type=F32)
    m = _silu(t)
    m = _silu(jnp.dot(m, w2b_ref[:], preferred_element_type=F32)
              + be2b_ref[:])
    c = _silu(jnp.dot(m, w1b_ref[:], preferred_element_type=F32)
              + bc1b_ref[:])
    qb = jnp.dot(c, wq2_ref[:], preferred_element_type=F32)
    rba = pltpu.roll(g, 125, 1)
    s = jnp.tanh(qb) * 15.0 / (jnp.sqrt(rba) + 1.0)
    out = g * s
    vl_ref[:] = out[:, 32:32 + VW]
    vr_ref[:] = out[:, 96:96 + VW]


def _scatter_body(ei_hbm, vl_hbm, vr_hbm, zeros_hbm, p_hbm,
                  idx_r, vbuf, shared):
    npad = zeros_hbm.shape[0]
    rows_per_tile = npad // NS
    cid = lax.axis_index("c")
    sid = lax.axis_index("s")
    wid = sid * NC + cid
    ep2 = vl_hbm.shape[0]
    ecs = ep2 // NW
    n_chunks = ecs // CHS
    base = wid * ecs

    pltpu.sync_copy(zeros_hbm.at[pl.ds(sid * rows_per_tile, rows_per_tile)],
                    shared.at[pl.ds(sid * rows_per_tile, rows_per_tile)])
    plsc.subcore_barrier()

    def make_step(v_hbm, ibase):
        def step(i, carry):
            off = base + i * CHS
            pltpu.sync_copy(ei_hbm.at[0, pl.ds(ibase + off, CHS)], idx_r)
            pltpu.sync_copy(v_hbm.at[pl.ds(off, CHS)], vbuf)
            pltpu.sync_copy(vbuf, shared.at[idx_r], add=True)
            return carry
        return step

    lax.fori_loop(0, n_chunks, make_step(vl_hbm, 0), 0)
    lax.fori_loop(0, n_chunks, make_step(vr_hbm, ep2), 0)
    plsc.subcore_barrier()
    pltpu.sync_copy(shared.at[pl.ds(sid * rows_per_tile, rows_per_tile)],
                    p_hbm.at[cid, pl.ds(sid * rows_per_tile, rows_per_tile)])


def _combine_body(p_ref, l_ref, nn_ref, out_ref):
    lv = l_ref[0, 0]
    i = pl.program_id(0)
    blk = out_ref.shape[0]
    p = p_ref[:]
    s = p[0] + p[1]
    rowid = i * blk + lax.broadcasted_iota(I32, (blk, 1), 0)
    s = jnp.where(rowid < nn_ref[0, 0], s, jnp.zeros_like(s))
    s = jnp.where(s > 0.5 * lv, s - lv, s)
    s = jnp.where(s < -0.5 * lv, s + lv, s)
    out_ref[:] = s[:, :3]


def kernel(x, z, pos, edge_index, batch, num_nodes, l,
           W_emb, b_emb, We1, be1, We2, be2, Wc1, bc1, Wc2):
    n = x.shape[0]
    e = edge_index.shape[1]
    blkn = 1000
    lnode = 2000
    echunk = NW * CHG
    ep = ((e + echunk - 1) // echunk) * echunk
    ep2 = ep // 2
    if ep == e and n % lnode == 0:
        npad = n
    else:
        npad = (n // lnode + 1) * lnode

    if npad == n:
        x2 = x.astype(F32).reshape(n, 1)
        b2 = batch.astype(I32).reshape(n, 1)
        pos2 = pos.astype(F32)
    else:
        x2 = jnp.pad(x.astype(F32), (0, npad - n)).reshape(npad, 1)
        b2 = jnp.pad(batch.astype(I32), (0, npad - n)).reshape(npad, 1)
        pos2 = jnp.pad(pos.astype(F32), ((0, npad - n), (0, 0)))
    ei = edge_index.astype(I32)
    if ep != e:
        ei = jnp.concatenate(
            [ei, jnp.full((2, ep - e), npad - 1, I32)], axis=1)
    l11 = jnp.asarray(l, F32).reshape(1, 1)
    lvec = jnp.full((16,), l, F32)
    nn11 = jnp.asarray(num_nodes, I32).reshape(1, 1)
    b_emb2 = b_emb.reshape(1, -1).astype(F32)
    be12 = be1.reshape(1, -1).astype(F32)
    we1r = We1[64, :].astype(F32)

    grid_n = npad // blkn
    ta, tb = pl.pallas_call(
        _node_body,
        grid=(grid_n,),
        in_specs=[
            pl.BlockSpec((blkn, 1), lambda i: (i, 0)),
            pl.BlockSpec((blkn, 1), lambda i: (i, 0)),
            pl.BlockSpec((blkn, 3), lambda i: (i, 0)),
            pl.BlockSpec(z.shape, lambda i: (0, 0)),
            pl.BlockSpec(W_emb.shape, lambda i: (0, 0)),
            pl.BlockSpec((1, 32), lambda i: (0, 0)),
            pl.BlockSpec(We1.shape, lambda i: (0, 0)),
            pl.BlockSpec((1, 32), lambda i: (0, 0)),
        ],
        out_specs=[
            pl.BlockSpec((blkn, TW), lambda i: (i, 0)),
            pl.BlockSpec((blkn, TW), lambda i: (i, 0)),
        ],
        out_shape=[
            jax.ShapeDtypeStruct((npad, TW), F32),
            jax.ShapeDtypeStruct((npad, TW), F32),
        ],
    )(x2, b2, pos2, z.astype(F32), W_emb.astype(F32), b_emb2,
      We1.astype(F32), be12)

    mesh = plsc.VectorSubcoreMesh(core_axis_name="c", subcore_axis_name="s")
    gather = pl.kernel(
        _gather_body,
        out_type=jax.ShapeDtypeStruct((ep2, 128), F32),
        mesh=mesh,
        compiler_params=pltpu.CompilerParams(use_tc_tiling_on_sc=False,
                                             needs_layout_passes=False),
        scratch_types=[
            pltpu.VMEM((CHG,), I32),
            pltpu.VMEM((CHG,), I32),
            pltpu.VMEM((CHG, TW), F32),
            pltpu.VMEM((16,), F32),
            pltpu.SemaphoreType.DMA,
        ],
    )
    g2 = gather(ei, ta, tb, lvec)

    blk2 = 3200
    egrid = ep2 // blk2
    We2f = We2.astype(F32)
    Wc1f = Wc1.astype(F32)
    Wc2f = Wc2.astype(F32)
    sr = (jnp.zeros((128, 128), F32)
          .at[35, 0:32].set(we1r).at[99, 64:96].set(we1r))
    w2b = (jnp.zeros((128, 128), F32)
           .at[0:32, 0:32].set(We2f).at[64:96, 64:96].set(We2f))
    be2b = (jnp.zeros((1, 128), F32)
            .at[0, 0:32].set(be2.astype(F32))
            .at[0, 64:96].set(be2.astype(F32)))
    w1b = (jnp.zeros((128, 128), F32)
           .at[0:32, 0:32].set(Wc1f).at[64:96, 64:96].set(Wc1f))
    bc1b = (jnp.zeros((1, 128), F32)
            .at[0, 0:32].set(bc1.astype(F32))
            .at[0, 64:96].set(bc1.astype(F32)))
    wq2 = (jnp.zeros((128, 128), F32)
           .at[0:32, 32:35].set(jnp.tile(Wc2f, (1, 3)))
           .at[64:96, 96:99].set(jnp.tile(Wc2f, (1, 3))))
    vl, vr = pl.pallas_call(
        _mlp_body,
        grid=(egrid,),
        in_specs=[
            pl.BlockSpec((blk2, 128), lambda i: (i, 0)),
            pl.BlockSpec((128, 128), lambda i: (0, 0)),
            pl.BlockSpec((128, 128), lambda i: (0, 0)),
            pl.BlockSpec((1, 128), lambda i: (0, 0)),
            pl.BlockSpec((128, 128), lambda i: (0, 0)),
            pl.BlockSpec((1, 128), lambda i: (0, 0)),
            pl.BlockSpec((128, 128), lambda i: (0, 0)),
        ],
        out_specs=[
            pl.BlockSpec((blk2, VW), lambda i: (i, 0)),
            pl.BlockSpec((blk2, VW), lambda i: (i, 0)),
        ],
        out_shape=[
            jax.ShapeDtypeStruct((ep2, VW), F32),
            jax.ShapeDtypeStruct((ep2, VW), F32),
        ],
    )(g2, sr, w2b, be2b, w1b, bc1b, wq2)

    zeros_tab = jnp.zeros((npad, VW), F32)
    scatter = pl.kernel(
        _scatter_body,
        out_type=jax.ShapeDtypeStruct((NC, npad, VW), F32),
        mesh=plsc.VectorSubcoreMesh(core_axis_name="c", subcore_axis_name="s"),
        compiler_params=pltpu.CompilerParams(use_tc_tiling_on_sc=False),
        scratch_types=[
            pltpu.VMEM((CHS,), I32),
            pltpu.VMEM((CHS, VW), F32),
            pltpu.VMEM_SHARED((npad, VW), F32),
        ],
    )
    p = scatter(ei, vl, vr, zeros_tab)

    diff = pl.pallas_call(
        _combine_body,
        grid=(npad // blkn,),
        in_specs=[
            pl.BlockSpec((NC, blkn, VW), lambda i: (0, i, 0)),
            pl.BlockSpec(memory_space=pltpu.SMEM),
            pl.BlockSpec(memory_space=pltpu.SMEM),
        ],
        out_specs=pl.BlockSpec((blkn, 3), lambda i: (i, 0)),
        out_shape=jax.ShapeDtypeStruct((npad, 3), F32),
    )(p, l11, nn11)

    return diff if npad == n else diff[:n]
